# baseline (speedup 1.0000x reference)
import functools

import jax
import jax.numpy as jnp
from jax import lax
from jax.experimental import pallas as pl
from jax.experimental.pallas import tpu as pltpu
from jax.experimental.pallas import tpu_sc as plsc

NUM_CORES = 2
NUM_SUBCORES = 16
NUM_WORKERS = NUM_CORES * NUM_SUBCORES
IDX_CHUNK = 128

_B = 16384
_V = 1000
_VP = 1024
_D = 128
_B_SC = 8192
_B_TC = _B - _B_SC
_BB = 1024


def _make_sc_lookup(D, B):
  assert B % (NUM_WORKERS * IDX_CHUNK) == 0
  b_per_w = B // NUM_WORKERS
  n_chunks = b_per_w // IDX_CHUNK
  mesh = plsc.VectorSubcoreMesh(core_axis_name="c", subcore_axis_name="s")

  @functools.partial(
      pl.kernel,
      mesh=mesh,
      out_type=jax.ShapeDtypeStruct((B, D), jnp.float32),
      scratch_types=[
          pltpu.VMEM((n_chunks, IDX_CHUNK), jnp.int32),
          pltpu.VMEM((b_per_w, D), jnp.float32),
      ] + [pltpu.SemaphoreType.DMA] * (n_chunks + 1),
  )
  def lookup(table_hbm, idx_hbm, out_hbm, idx_v, rows_v, *sems):
    g_sems, w_sem = sems[:n_chunks], sems[n_chunks]
    wid = lax.axis_index("s") * NUM_CORES + lax.axis_index("c")
    base = wid * b_per_w
    pltpu.sync_copy(idx_hbm.at[wid], idx_v)
    gathers = []
    writes = []
    for j in range(n_chunks):
      gathers.append(
          pltpu.async_copy(
              table_hbm.at[idx_v.at[j]],
              rows_v.at[pl.ds(j * IDX_CHUNK, IDX_CHUNK)],
              g_sems[j],
          ))
    for j in range(n_chunks):
      gathers[j].wait()
      writes.append(
          pltpu.async_copy(
              rows_v.at[pl.ds(j * IDX_CHUNK, IDX_CHUNK)],
              out_hbm.at[pl.ds(base + j * IDX_CHUNK, IDX_CHUNK)],
              w_sem,
          ))
    for w in writes:
      w.wait()

  return lookup


_SC_LOOKUP = _make_sc_lookup(_D, _B_SC)


def _tc_body(ids_ref, table_ref, out_ref):
  ids = ids_ref[...]
  iota = lax.broadcasted_iota(jnp.int32, (_BB, _VP), 1)
  oh = (iota == ids).astype(jnp.bfloat16)
  t = table_ref[...]
  t_hi = t.astype(jnp.bfloat16)
  t_lo = (t - t_hi.astype(jnp.float32)).astype(jnp.bfloat16)
  out_ref[...] = (
      jnp.dot(oh, t_hi, preferred_element_type=jnp.float32)
      + jnp.dot(oh, t_lo, preferred_element_type=jnp.float32))


_tc_gather = pl.pallas_call(
    _tc_body,
    grid=(_B_TC // _BB,),
    in_specs=[
        pl.BlockSpec((_BB, 1), lambda i: (i, 0)),
        pl.BlockSpec((_VP, _D), lambda i: (0, 0)),
    ],
    out_specs=pl.BlockSpec((_BB, _D), lambda i: (i, 0)),
    out_shape=jax.ShapeDtypeStruct((_B_TC, _D), jnp.float32),
)


@jax.jit
def kernel(violation_ids, violation_embedding):
  ids32 = violation_ids.astype(jnp.int32)
  idx_sc = ids32[:_B_SC].reshape(
      NUM_WORKERS, _B_SC // NUM_WORKERS // IDX_CHUNK, IDX_CHUNK)
  out_sc = _SC_LOOKUP(violation_embedding, idx_sc)
  table_p = jnp.pad(violation_embedding, ((0, _VP - _V), (0, 0)))
  out_tc = _tc_gather(ids32[_B_SC:].reshape(_B_TC, 1), table_p)
  return jnp.concatenate([out_sc, out_tc], axis=0)


# speedup vs baseline: 1.4885x; 1.4885x over previous
import functools

import jax
import jax.numpy as jnp
from jax import lax
from jax.experimental import pallas as pl
from jax.experimental.pallas import tpu as pltpu
from jax.experimental.pallas import tpu_sc as plsc

NUM_CORES = 2
NUM_SUBCORES = 16
NUM_WORKERS = NUM_CORES * NUM_SUBCORES
IDX_CHUNK = 128


def _make_lookup(D, B):
  assert B % (NUM_WORKERS * IDX_CHUNK) == 0
  b_per_w = B // NUM_WORKERS
  n_chunks = b_per_w // IDX_CHUNK
  mesh = plsc.VectorSubcoreMesh(core_axis_name="c", subcore_axis_name="s")

  @functools.partial(
      pl.kernel,
      mesh=mesh,
      out_type=jax.ShapeDtypeStruct((B, D), jnp.float32),
      scratch_types=[
          pltpu.VMEM((b_per_w,), jnp.int32),
          pltpu.VMEM((b_per_w, D), jnp.float32),
          pltpu.SemaphoreType.DMA,
      ],
  )
  def lookup(table_hbm, idx_hbm, out_hbm, idx_v, rows_v, sem):
    wid = lax.axis_index("s") * NUM_CORES + lax.axis_index("c")
    base = wid * b_per_w
    pltpu.sync_copy(idx_hbm.at[pl.ds(base, b_per_w)], idx_v)
    copies = []
    for j in range(n_chunks):
      copies.append(
          pltpu.async_copy(
              table_hbm.at[idx_v.at[pl.ds(j * IDX_CHUNK, IDX_CHUNK)]],
              rows_v.at[pl.ds(j * IDX_CHUNK, IDX_CHUNK)],
              sem,
          ))
    for c in copies:
      c.wait()
    pltpu.sync_copy(rows_v, out_hbm.at[pl.ds(base, b_per_w)])

  return lookup


_LOOKUP = _make_lookup(128, 16384)


@jax.jit
def kernel(violation_ids, violation_embedding):
  return _LOOKUP(violation_embedding, violation_ids.astype(jnp.int32))


# speedup vs baseline: 1.4899x; 1.0009x over previous
import functools

import jax
import jax.numpy as jnp
from jax import lax
from jax.experimental import pallas as pl
from jax.experimental.pallas import tpu as pltpu
from jax.experimental.pallas import tpu_sc as plsc

NUM_CORES = 2
NUM_SUBCORES = 16
NUM_WORKERS = NUM_CORES * NUM_SUBCORES
IDX_CHUNK = 128


def _make_lookup(D, B):
  assert B % (NUM_WORKERS * IDX_CHUNK) == 0
  b_per_w = B // NUM_WORKERS
  n_chunks = b_per_w // IDX_CHUNK
  mesh = plsc.VectorSubcoreMesh(core_axis_name="c", subcore_axis_name="s")

  @functools.partial(
      pl.kernel,
      mesh=mesh,
      out_type=jax.ShapeDtypeStruct((B, D), jnp.float32),
      scratch_types=[
          pltpu.VMEM((b_per_w,), jnp.int32),
          pltpu.VMEM((b_per_w, D), jnp.float32),
          pltpu.SemaphoreType.DMA,
      ] + [pltpu.SemaphoreType.DMA] * n_chunks,
  )
  def lookup(table_hbm, idx_hbm, out_hbm, idx_v, rows_v, sem, *i_sems):
    wid = lax.axis_index("s") * NUM_CORES + lax.axis_index("c")
    base = wid * b_per_w
    stages = []
    for j in range(n_chunks):
      stages.append(
          pltpu.async_copy(
              idx_hbm.at[pl.ds(base + j * IDX_CHUNK, IDX_CHUNK)],
              idx_v.at[pl.ds(j * IDX_CHUNK, IDX_CHUNK)],
              i_sems[j],
          ))
    copies = []
    for j in range(n_chunks):
      stages[j].wait()
      copies.append(
          pltpu.async_copy(
              table_hbm.at[idx_v.at[pl.ds(j * IDX_CHUNK, IDX_CHUNK)]],
              rows_v.at[pl.ds(j * IDX_CHUNK, IDX_CHUNK)],
              sem,
          ))
    for c in copies:
      c.wait()
    pltpu.sync_copy(rows_v, out_hbm.at[pl.ds(base, b_per_w)])

  return lookup


_LOOKUP = _make_lookup(128, 16384)


@jax.jit
def kernel(violation_ids, violation_embedding):
  return _LOOKUP(violation_embedding, violation_ids.astype(jnp.int32))
